# baseline (speedup 1.0000x reference)
import jax
import jax.numpy as jnp
from jax import lax
from jax.experimental import pallas as pl
from jax.experimental.pallas import tpu as pltpu
from jax.experimental.pallas import tpu_sc as plsc

_B, _L, _D = 64, 256, 1024
_HALF = 128
_MARGINS = (0.2, 0.4)
_NSUB = 16
_LANES = 16
_ROWS_PER_TILE = _B // _NSUB
_NCHUNK = _HALF // _LANES


def _sim_body(t_ref, l_ref, tgt_ref, out_ref):
    t = t_ref[...]
    lbl = l_ref[...]
    tn = t / jnp.maximum(jnp.sqrt(jnp.sum(t * t, axis=-1, keepdims=True)), 1e-12)
    ln = lbl / jnp.maximum(jnp.sqrt(jnp.sum(lbl * lbl, axis=-1, keepdims=True)), 1e-12)
    out_ref[pl.ds(0, _B), :] = jax.lax.dot_general(
        tn, ln, (((1,), (1,)), ((), ())),
        preferred_element_type=jnp.float32,
        precision=jax.lax.Precision.HIGHEST,
    )
    out_ref[pl.ds(_B, _B), :] = jnp.where(tgt_ref[...] != 0, 1.0, 0.0)


def _sc_mine_body(packed_hbm, out_hbm, part_hbm, sim_v, vm_v, vec_v, acc_v,
                  red_v, sem1, sem2):
    sid = lax.axis_index("s")
    cid = lax.axis_index("c")

    @pl.when(cid == 0)
    def _core0():
        iota = lax.iota(jnp.int32, _LANES)
        zeros = jnp.zeros((_LANES,), jnp.float32)

        cp1 = pltpu.async_copy(
            packed_hbm.at[pl.ds(sid * _ROWS_PER_TILE, _ROWS_PER_TILE)],
            sim_v, sem1)
        cp2 = pltpu.async_copy(
            packed_hbm.at[pl.ds(_B + sid * _ROWS_PER_TILE, _ROWS_PER_TILE)],
            vm_v, sem2)
        for i in range(4):
            acc_v[i, :] = zeros
        cp1.wait()
        cp2.wait()

        @pl.loop(0, _ROWS_PER_TILE)
        def _row(r):
            for lvl in range(2):
                base = lvl * _HALF
                margin = _MARGINS[lvl]

                mv0 = sim_v[r, pl.ds(base, _LANES)]
                mv1 = jnp.maximum(sim_v[r, pl.ds(base + _LANES, _LANES)],
                                  sim_v[r, pl.ds(base + 2 * _LANES, _LANES)])
                mv2 = jnp.maximum(sim_v[r, pl.ds(base + 3 * _LANES, _LANES)],
                                  sim_v[r, pl.ds(base + 4 * _LANES, _LANES)])
                mv3 = jnp.maximum(sim_v[r, pl.ds(base + 5 * _LANES, _LANES)],
                                  sim_v[r, pl.ds(base + 6 * _LANES, _LANES)])
                mv = jnp.maximum(
                    jnp.maximum(mv0, sim_v[r, pl.ds(base + 7 * _LANES, _LANES)]),
                    jnp.maximum(jnp.maximum(mv1, mv2), mv3))
                m = jnp.max(mv)

                c0 = mv0
                s0 = jnp.sum(jnp.where(iota == 0, c0, 0.0))
                s1 = jnp.sum(jnp.where(iota == 1, c0, 0.0))
                mm = m + margin

                fb0 = jnp.where(iota == 0, s1, s0) + margin
                negv = jnp.where(c0 < m, mm, fb0)
                vm = vm_v[r, pl.ds(base, _LANES)]
                plsc.addupdate(acc_v.at[2 * lvl, :],
                               jnp.maximum(negv - c0, 0.0) * vm)
                plsc.addupdate(acc_v.at[2 * lvl + 1, :], vm)

                fbs = s0 + margin

                @pl.loop(1, _NCHUNK)
                def _chunk(k):
                    off = base + k * _LANES
                    ck = sim_v[r, pl.ds(off, _LANES)]
                    negvk = jnp.where(ck < m, mm, fbs)
                    vmk = vm_v[r, pl.ds(off, _LANES)]
                    plsc.addupdate(acc_v.at[2 * lvl, :],
                                   jnp.maximum(negvk - ck, 0.0) * vmk)
                    plsc.addupdate(acc_v.at[2 * lvl + 1, :], vmk)

        part = (jnp.where(iota == 0, jnp.sum(acc_v[0, :]), 0.0)
                + jnp.where(iota == 1, jnp.sum(acc_v[1, :]), 0.0)
                + jnp.where(iota == 2, jnp.sum(acc_v[2, :]), 0.0)
                + jnp.where(iota == 3, jnp.sum(acc_v[3, :]), 0.0))
        vec_v[...] = part
        pltpu.sync_copy(vec_v, part_hbm.at[sid])
        plsc.subcore_barrier()

        @pl.when(sid == 0)
        def _():
            pltpu.sync_copy(part_hbm, red_v)
            acc = red_v[0, :]
            for i in range(1, _NSUB):
                acc = acc + red_v[i, :]
            big_s1 = jnp.sum(jnp.where(iota == 0, acc, 0.0))
            big_c1 = jnp.sum(jnp.where(iota == 1, acc, 0.0))
            big_s2 = jnp.sum(jnp.where(iota == 2, acc, 0.0))
            big_c2 = jnp.sum(jnp.where(iota == 3, acc, 0.0))
            nv = (jnp.where(iota == 0, big_s1, 0.0)
                  + jnp.where(iota == 1, big_s2, 0.0))
            c2v = jnp.where(iota >= 0, big_c2, 0.0)
            dv = jnp.where(iota == 0, big_c1, jnp.maximum(c2v, 1.0))
            q = nv / dv
            gate = jnp.where(
                (iota == 0) | ((iota == 1) & (c2v >= 3.0)), 1.0, 0.0)
            loss = jnp.sum(q * gate)
            vec_v[...] = jnp.where(iota == 0, loss, 0.0)
            pltpu.sync_copy(vec_v, out_hbm.at[cid])


def kernel(text_embed, label_embed, target):
    tgt = target.astype(jnp.int32)
    packed = pl.pallas_call(
        _sim_body,
        out_shape=jax.ShapeDtypeStruct((2 * _B, _L), jnp.float32),
    )(text_embed, label_embed, tgt)

    mine = pl.kernel(
        _sc_mine_body,
        out_type=(jax.ShapeDtypeStruct((2, _LANES), jnp.float32),
                  jax.ShapeDtypeStruct((_NSUB, _LANES), jnp.float32)),
        mesh=plsc.VectorSubcoreMesh(
            core_axis_name="c", subcore_axis_name="s",
            num_cores=2, num_subcores=_NSUB),
        scratch_types=[
            pltpu.VMEM((_ROWS_PER_TILE, _L), jnp.float32),
            pltpu.VMEM((_ROWS_PER_TILE, _L), jnp.float32),
            pltpu.VMEM((_LANES,), jnp.float32),
            pltpu.VMEM((4, _LANES), jnp.float32),
            pltpu.VMEM((_NSUB, _LANES), jnp.float32),
            pltpu.SemaphoreType.DMA,
            pltpu.SemaphoreType.DMA,
        ],
        compiler_params=pltpu.CompilerParams(needs_layout_passes=False),
    )
    out, _ = mine(packed)
    return out[0, 0]


# speedup vs baseline: 1.0246x; 1.0246x over previous
import jax
import jax.numpy as jnp
from jax import lax
from jax.experimental import pallas as pl
from jax.experimental.pallas import tpu as pltpu
from jax.experimental.pallas import tpu_sc as plsc

_B, _L, _D = 64, 256, 1024
_HALF = 128
_MARGINS = (0.2, 0.4)
_NSUB = 16
_LANES = 16
_ROWS_PER_TILE = _B // _NSUB
_NCHUNK = _HALF // _LANES


def _sim_body(t_ref, l_ref, tgt_ref, out_ref):
    t = t_ref[...]
    lbl = l_ref[...]
    tn = t / jnp.maximum(jnp.sqrt(jnp.sum(t * t, axis=-1, keepdims=True)), 1e-12)
    ln = lbl / jnp.maximum(jnp.sqrt(jnp.sum(lbl * lbl, axis=-1, keepdims=True)), 1e-12)
    out_ref[pl.ds(0, _B), :] = jax.lax.dot_general(
        tn, ln, (((1,), (1,)), ((), ())),
        preferred_element_type=jnp.float32,
        precision=jax.lax.Precision.HIGHEST,
    )
    out_ref[pl.ds(_B, _B), :] = jnp.where(tgt_ref[...] != 0, 1.0, 0.0)


def _sc_mine_body(packed_hbm, out_hbm, sim_v, vm_v, vec_v, loss_v, red_v,
                  sem1, sem2):
    sid = lax.axis_index("s")
    cid = lax.axis_index("c")

    @pl.when(cid == 0)
    def _core0():
        iota = lax.iota(jnp.int32, _LANES)
        zeros = jnp.zeros((_LANES,), jnp.float32)

        cp1 = pltpu.async_copy(
            packed_hbm.at[pl.ds(sid * _ROWS_PER_TILE, _ROWS_PER_TILE)],
            sim_v, sem1)
        cp2 = pltpu.async_copy(
            packed_hbm.at[pl.ds(_B + sid * _ROWS_PER_TILE, _ROWS_PER_TILE)],
            vm_v, sem2)
        cp1.wait()
        cp2.wait()

        def per_row(r, accs):
            new = []
            for lvl in range(2):
                acc_s, acc_c = accs[2 * lvl], accs[2 * lvl + 1]
                base = lvl * _HALF
                margin = _MARGINS[lvl]

                c0 = sim_v[r, pl.ds(base, _LANES)]

                def max_step(k, mv):
                    return jnp.maximum(
                        mv, sim_v[r, pl.ds(base + k * _LANES, _LANES)])
                mv = lax.fori_loop(1, _NCHUNK, max_step, c0)
                m = jnp.max(mv)

                s0 = jnp.sum(jnp.where(iota == 0, c0, 0.0))
                s1 = jnp.sum(jnp.where(iota == 1, c0, 0.0))
                mm = m + margin

                fb0 = jnp.where(iota == 0, s1, s0) + margin
                negv = jnp.where(c0 < m, mm, fb0)
                vm0 = vm_v[r, pl.ds(base, _LANES)]
                acc_s = acc_s + jnp.maximum(negv - c0, 0.0) * vm0
                acc_c = acc_c + vm0

                fbs = s0 + margin

                def hinge_step(k, sc):
                    a_s, a_c = sc
                    off = base + k * _LANES
                    ck = sim_v[r, pl.ds(off, _LANES)]
                    negvk = jnp.where(ck < m, mm, fbs)
                    vmk = vm_v[r, pl.ds(off, _LANES)]
                    return (a_s + jnp.maximum(negvk - ck, 0.0) * vmk,
                            a_c + vmk)
                acc_s, acc_c = lax.fori_loop(
                    1, _NCHUNK, hinge_step, (acc_s, acc_c))
                new.extend([acc_s, acc_c])
            return tuple(new)

        accs = lax.fori_loop(0, _ROWS_PER_TILE, per_row,
                             (zeros, zeros, zeros, zeros))

        part = (jnp.where(iota == 0, jnp.sum(accs[0]), 0.0)
                + jnp.where(iota == 1, jnp.sum(accs[1]), 0.0)
                + jnp.where(iota == 2, jnp.sum(accs[2]), 0.0)
                + jnp.where(iota == 3, jnp.sum(accs[3]), 0.0))
        vec_v[...] = part
        pltpu.sync_copy(vec_v, out_hbm.at[sid])
        plsc.subcore_barrier()

        @pl.when(sid == 0)
        def _():
            pltpu.sync_copy(out_hbm.at[pl.ds(0, _NSUB)], red_v)

            def red_step(i, acc):
                return acc + red_v[i, :]
            acc = lax.fori_loop(1, _NSUB, red_step, red_v[0, :])
            big_s1 = jnp.sum(jnp.where(iota == 0, acc, 0.0))
            big_c1 = jnp.sum(jnp.where(iota == 1, acc, 0.0))
            big_s2 = jnp.sum(jnp.where(iota == 2, acc, 0.0))
            big_c2 = jnp.sum(jnp.where(iota == 3, acc, 0.0))
            nv = (jnp.where(iota == 0, big_s1, 0.0)
                  + jnp.where(iota == 1, big_s2, 0.0))
            c2v = jnp.where(iota >= 0, big_c2, 0.0)
            dv = jnp.where(iota == 0, big_c1, jnp.maximum(c2v, 1.0))
            q = nv / dv
            gate = jnp.where(
                (iota == 0) | ((iota == 1) & (c2v >= 3.0)), 1.0, 0.0)
            loss = jnp.sum(q * gate)
            loss_v[0, :] = jnp.where(iota == 0, loss, 0.0)
            pltpu.sync_copy(loss_v, out_hbm.at[pl.ds(_NSUB, 1)])


def kernel(text_embed, label_embed, target):
    tgt = target.astype(jnp.int32)
    packed = pl.pallas_call(
        _sim_body,
        out_shape=jax.ShapeDtypeStruct((2 * _B, _L), jnp.float32),
    )(text_embed, label_embed, tgt)

    mine = pl.kernel(
        _sc_mine_body,
        out_type=jax.ShapeDtypeStruct((1 + _NSUB, _LANES), jnp.float32),
        mesh=plsc.VectorSubcoreMesh(
            core_axis_name="c", subcore_axis_name="s",
            num_cores=2, num_subcores=_NSUB),
        scratch_types=[
            pltpu.VMEM((_ROWS_PER_TILE, _L), jnp.float32),
            pltpu.VMEM((_ROWS_PER_TILE, _L), jnp.float32),
            pltpu.VMEM((_LANES,), jnp.float32),
            pltpu.VMEM((1, _LANES), jnp.float32),
            pltpu.VMEM((_NSUB, _LANES), jnp.float32),
            pltpu.SemaphoreType.DMA,
            pltpu.SemaphoreType.DMA,
        ],
        compiler_params=pltpu.CompilerParams(needs_layout_passes=False),
    )
    out = mine(packed)
    return out[_NSUB, 0]


# speedup vs baseline: 1.0431x; 1.0180x over previous
import jax
import jax.numpy as jnp
from jax import lax
from jax.experimental import pallas as pl
from jax.experimental.pallas import tpu as pltpu
from jax.experimental.pallas import tpu_sc as plsc

_B, _L, _D = 64, 256, 1024
_HALF = 128
_MARGINS = (0.2, 0.4)
_NSUB = 16
_LANES = 16
_ROWS_PER_TILE = _B // _NSUB
_NCHUNK = _HALF // _LANES


def _sim_body(t_ref, l_ref, tgt_ref, out_ref):
    t = t_ref[...]
    lbl = l_ref[...]
    tn = t / jnp.maximum(jnp.sqrt(jnp.sum(t * t, axis=-1, keepdims=True)), 1e-12)
    ln = lbl / jnp.maximum(jnp.sqrt(jnp.sum(lbl * lbl, axis=-1, keepdims=True)), 1e-12)
    out_ref[pl.ds(0, _B), :] = jax.lax.dot_general(
        tn, ln, (((1,), (1,)), ((), ())),
        preferred_element_type=jnp.float32,
        precision=jax.lax.Precision.DEFAULT,
    )
    out_ref[pl.ds(_B, _B), :] = jnp.where(tgt_ref[...] != 0, 1.0, 0.0)


def _sc_mine_body(packed_hbm, out_hbm, sim_v, vm_v, vec_v, loss_v, red_v,
                  sem1, sem2):
    sid = lax.axis_index("s")
    cid = lax.axis_index("c")

    @pl.when(cid == 0)
    def _core0():
        iota = lax.iota(jnp.int32, _LANES)
        zeros = jnp.zeros((_LANES,), jnp.float32)

        cp1 = pltpu.async_copy(
            packed_hbm.at[pl.ds(sid * _ROWS_PER_TILE, _ROWS_PER_TILE)],
            sim_v, sem1)
        cp2 = pltpu.async_copy(
            packed_hbm.at[pl.ds(_B + sid * _ROWS_PER_TILE, _ROWS_PER_TILE)],
            vm_v, sem2)
        cp1.wait()
        cp2.wait()

        def per_row(r, accs):
            new = []
            for lvl in range(2):
                acc_s, acc_c = accs[2 * lvl], accs[2 * lvl + 1]
                base = lvl * _HALF
                margin = _MARGINS[lvl]

                c0 = sim_v[r, pl.ds(base, _LANES)]

                def max_step(k, mv):
                    return jnp.maximum(
                        mv, sim_v[r, pl.ds(base + k * _LANES, _LANES)])
                mv = lax.fori_loop(1, _NCHUNK, max_step, c0)
                m = jnp.max(mv)

                s0 = jnp.sum(jnp.where(iota == 0, c0, 0.0))
                s1 = jnp.sum(jnp.where(iota == 1, c0, 0.0))
                mm = m + margin

                fb0 = jnp.where(iota == 0, s1, s0) + margin
                negv = jnp.where(c0 < m, mm, fb0)
                vm0 = vm_v[r, pl.ds(base, _LANES)]
                acc_s = acc_s + jnp.maximum(negv - c0, 0.0) * vm0
                acc_c = acc_c + vm0

                fbs = s0 + margin

                def hinge_step(k, sc):
                    a_s, a_c = sc
                    off = base + k * _LANES
                    ck = sim_v[r, pl.ds(off, _LANES)]
                    negvk = jnp.where(ck < m, mm, fbs)
                    vmk = vm_v[r, pl.ds(off, _LANES)]
                    return (a_s + jnp.maximum(negvk - ck, 0.0) * vmk,
                            a_c + vmk)
                acc_s, acc_c = lax.fori_loop(
                    1, _NCHUNK, hinge_step, (acc_s, acc_c))
                new.extend([acc_s, acc_c])
            return tuple(new)

        accs = lax.fori_loop(0, _ROWS_PER_TILE, per_row,
                             (zeros, zeros, zeros, zeros))

        part = (jnp.where(iota == 0, jnp.sum(accs[0]), 0.0)
                + jnp.where(iota == 1, jnp.sum(accs[1]), 0.0)
                + jnp.where(iota == 2, jnp.sum(accs[2]), 0.0)
                + jnp.where(iota == 3, jnp.sum(accs[3]), 0.0))
        vec_v[...] = part
        pltpu.sync_copy(vec_v, out_hbm.at[sid])
        plsc.subcore_barrier()

        @pl.when(sid == 0)
        def _():
            pltpu.sync_copy(out_hbm.at[pl.ds(0, _NSUB)], red_v)

            def red_step(i, acc):
                return acc + red_v[i, :]
            acc = lax.fori_loop(1, _NSUB, red_step, red_v[0, :])
            big_s1 = jnp.sum(jnp.where(iota == 0, acc, 0.0))
            big_c1 = jnp.sum(jnp.where(iota == 1, acc, 0.0))
            big_s2 = jnp.sum(jnp.where(iota == 2, acc, 0.0))
            big_c2 = jnp.sum(jnp.where(iota == 3, acc, 0.0))
            nv = (jnp.where(iota == 0, big_s1, 0.0)
                  + jnp.where(iota == 1, big_s2, 0.0))
            c2v = jnp.where(iota >= 0, big_c2, 0.0)
            dv = jnp.where(iota == 0, big_c1, jnp.maximum(c2v, 1.0))
            q = nv / dv
            gate = jnp.where(
                (iota == 0) | ((iota == 1) & (c2v >= 3.0)), 1.0, 0.0)
            loss = jnp.sum(q * gate)
            loss_v[0, :] = jnp.where(iota == 0, loss, 0.0)
            pltpu.sync_copy(loss_v, out_hbm.at[pl.ds(_NSUB, 1)])


def kernel(text_embed, label_embed, target):
    tgt = target.astype(jnp.int32)
    packed = pl.pallas_call(
        _sim_body,
        out_shape=jax.ShapeDtypeStruct((2 * _B, _L), jnp.float32),
    )(text_embed, label_embed, tgt)

    mine = pl.kernel(
        _sc_mine_body,
        out_type=jax.ShapeDtypeStruct((1 + _NSUB, _LANES), jnp.float32),
        mesh=plsc.VectorSubcoreMesh(
            core_axis_name="c", subcore_axis_name="s",
            num_cores=2, num_subcores=_NSUB),
        scratch_types=[
            pltpu.VMEM((_ROWS_PER_TILE, _L), jnp.float32),
            pltpu.VMEM((_ROWS_PER_TILE, _L), jnp.float32),
            pltpu.VMEM((_LANES,), jnp.float32),
            pltpu.VMEM((1, _LANES), jnp.float32),
            pltpu.VMEM((_NSUB, _LANES), jnp.float32),
            pltpu.SemaphoreType.DMA,
            pltpu.SemaphoreType.DMA,
        ],
        compiler_params=pltpu.CompilerParams(needs_layout_passes=False),
    )
    out = mine(packed)
    return out[_NSUB, 0]


# speedup vs baseline: 1.1207x; 1.0744x over previous
import jax
import jax.numpy as jnp
from jax import lax
from jax.experimental import pallas as pl
from jax.experimental.pallas import tpu as pltpu
from jax.experimental.pallas import tpu_sc as plsc

_B, _L, _D = 64, 256, 1024
_HALF = 128
_MARGINS = (0.2, 0.4)
_NSUB = 16
_LANES = 16
_ROWS_PER_TILE = _B // _NSUB
_NCHUNK = _HALF // _LANES


def _sim_body(t_ref, l_ref, tgt_ref, out_ref):
    t = t_ref[...]
    lbl = l_ref[...]
    tn = t / jnp.maximum(jnp.sqrt(jnp.sum(t * t, axis=-1, keepdims=True)), 1e-12)
    ln = lbl / jnp.maximum(jnp.sqrt(jnp.sum(lbl * lbl, axis=-1, keepdims=True)), 1e-12)
    out_ref[pl.ds(0, _B), :] = jax.lax.dot_general(
        tn, ln, (((1,), (1,)), ((), ())),
        preferred_element_type=jnp.float32,
        precision=jax.lax.Precision.DEFAULT,
    )
    out_ref[pl.ds(_B, _B), :] = jnp.where(tgt_ref[...] != 0, 1.0, 0.0)


def _sc_mine_body(packed_hbm, out_hbm, sim_v, vm_v, vec_v, loss_v, red_v,
                  sem1, sem2):
    sid = lax.axis_index("s")
    cid = lax.axis_index("c")

    @pl.when(cid == 0)
    def _core0():
        iota = lax.iota(jnp.int32, _LANES)
        zeros = jnp.zeros((_LANES,), jnp.float32)

        cp1 = pltpu.async_copy(
            packed_hbm.at[pl.ds(sid * _ROWS_PER_TILE, _ROWS_PER_TILE)],
            sim_v, sem1)
        cp2 = pltpu.async_copy(
            packed_hbm.at[pl.ds(_B + sid * _ROWS_PER_TILE, _ROWS_PER_TILE)],
            vm_v, sem2)
        cp1.wait()
        cp2.wait()

        def per_row(r, accs):
            new = []
            for lvl in range(2):
                acc_s, acc_c = accs[2 * lvl], accs[2 * lvl + 1]
                base = lvl * _HALF
                margin = _MARGINS[lvl]

                c0 = sim_v[r, pl.ds(base, _LANES)]

                def max_step(k, mv):
                    return jnp.maximum(
                        mv, sim_v[r, pl.ds(base + k * _LANES, _LANES)])
                mv = lax.fori_loop(1, _NCHUNK, max_step, c0)
                m = jnp.max(mv)

                s0 = jnp.sum(jnp.where(iota == 0, c0, 0.0))
                s1 = jnp.sum(jnp.where(iota == 1, c0, 0.0))
                mm = m + margin

                fb0 = jnp.where(iota == 0, s1, s0) + margin
                negv = jnp.where(c0 < m, mm, fb0)
                vm0 = vm_v[r, pl.ds(base, _LANES)]
                acc_s = acc_s + jnp.maximum(negv - c0, 0.0) * vm0
                acc_c = acc_c + vm0

                fbs = s0 + margin

                def hinge_step(k, sc):
                    a_s, a_c = sc
                    off = base + k * _LANES
                    ck = sim_v[r, pl.ds(off, _LANES)]
                    negvk = jnp.where(ck < m, mm, fbs)
                    vmk = vm_v[r, pl.ds(off, _LANES)]
                    return (a_s + jnp.maximum(negvk - ck, 0.0) * vmk,
                            a_c + vmk)
                acc_s, acc_c = lax.fori_loop(
                    1, _NCHUNK, hinge_step, (acc_s, acc_c))
                new.extend([acc_s, acc_c])
            return tuple(new)

        accs = lax.fori_loop(0, _ROWS_PER_TILE, per_row,
                             (zeros, zeros, zeros, zeros))

        part = (jnp.where(iota == 0, jnp.sum(accs[0]), 0.0)
                + jnp.where(iota == 1, jnp.sum(accs[1]), 0.0)
                + jnp.where(iota == 2, jnp.sum(accs[2]), 0.0)
                + jnp.where(iota == 3, jnp.sum(accs[3]), 0.0))
        vec_v[...] = part
        pltpu.sync_copy(vec_v, out_hbm.at[sid])
        plsc.subcore_barrier()

        @pl.when(sid == 0)
        def _():
            pltpu.sync_copy(out_hbm.at[pl.ds(0, _NSUB)], red_v)

            def red_step(i, acc):
                return acc + red_v[i, :]
            acc = lax.fori_loop(1, _NSUB, red_step, red_v[0, :])
            big_s1 = jnp.sum(jnp.where(iota == 0, acc, 0.0))
            big_c1 = jnp.sum(jnp.where(iota == 1, acc, 0.0))
            big_s2 = jnp.sum(jnp.where(iota == 2, acc, 0.0))
            big_c2 = jnp.sum(jnp.where(iota == 3, acc, 0.0))
            nv = (jnp.where(iota == 0, big_s1, 0.0)
                  + jnp.where(iota == 1, big_s2, 0.0))
            c2v = jnp.where(iota >= 0, big_c2, 0.0)
            dv = jnp.where(iota == 0, big_c1, jnp.maximum(c2v, 1.0))
            q = nv / dv
            gate = jnp.where(
                (iota == 0) | ((iota == 1) & (c2v >= 3.0)), 1.0, 0.0)
            loss = jnp.sum(q * gate)
            loss_v[0, :] = jnp.where(iota == 0, loss, 0.0)
            pltpu.sync_copy(loss_v, out_hbm.at[pl.ds(_NSUB, 1)])


def kernel(text_embed, label_embed, target):
    tgt = target.astype(jnp.int32)
    packed = pl.pallas_call(
        _sim_body,
        out_shape=jax.ShapeDtypeStruct((2 * _B, _L), jnp.float32),
    )(text_embed, label_embed, tgt)

    mine = pl.kernel(
        _sc_mine_body,
        out_type=jax.ShapeDtypeStruct((1 + _NSUB, _LANES), jnp.float32),
        mesh=plsc.VectorSubcoreMesh(
            core_axis_name="c", subcore_axis_name="s",
            num_cores=1, num_subcores=_NSUB),
        scratch_types=[
            pltpu.VMEM((_ROWS_PER_TILE, _L), jnp.float32),
            pltpu.VMEM((_ROWS_PER_TILE, _L), jnp.float32),
            pltpu.VMEM((_LANES,), jnp.float32),
            pltpu.VMEM((1, _LANES), jnp.float32),
            pltpu.VMEM((_NSUB, _LANES), jnp.float32),
            pltpu.SemaphoreType.DMA,
            pltpu.SemaphoreType.DMA,
        ],
        compiler_params=pltpu.CompilerParams(needs_layout_passes=False),
    )
    out = mine(packed)
    return out[_NSUB, 0]


# speedup vs baseline: 1.2277x; 1.0954x over previous
import jax
import jax.numpy as jnp
from jax import lax
from jax.experimental import pallas as pl
from jax.experimental.pallas import tpu as pltpu
from jax.experimental.pallas import tpu_sc as plsc

_B, _L, _D = 64, 256, 1024
_HALF = 128
_MARGINS = (0.2, 0.4)
_NSUB = 16
_LANES = 16
_ROWS_PER_TILE = _B // _NSUB
_NCHUNK = _HALF // _LANES


def _sim_body(t_ref, l_ref, tgt_ref, out_ref):
    t = t_ref[...]
    lbl = l_ref[...]
    tn = t / jnp.maximum(jnp.sqrt(jnp.sum(t * t, axis=-1, keepdims=True)), 1e-12)
    ln = lbl / jnp.maximum(jnp.sqrt(jnp.sum(lbl * lbl, axis=-1, keepdims=True)), 1e-12)
    out_ref[pl.ds(0, _B), :] = jax.lax.dot_general(
        tn, ln, (((1,), (1,)), ((), ())),
        preferred_element_type=jnp.float32,
        precision=jax.lax.Precision.DEFAULT,
    )
    out_ref[pl.ds(_B, _B), :] = jnp.where(tgt_ref[...] != 0, 1.0, 0.0)


def _sc_mine_body(packed_hbm, out_hbm, sim_v, vm_v, vec_v, loss_v, red_v,
                  shared, sem1, sem2):
    sid = lax.axis_index("s")
    cid = lax.axis_index("c")

    @pl.when(cid == 0)
    def _core0():
        iota = lax.iota(jnp.int32, _LANES)
        zeros = jnp.zeros((_LANES,), jnp.float32)

        cp1 = pltpu.async_copy(
            packed_hbm.at[pl.ds(sid * _ROWS_PER_TILE, _ROWS_PER_TILE)],
            sim_v, sem1)
        cp2 = pltpu.async_copy(
            packed_hbm.at[pl.ds(_B + sid * _ROWS_PER_TILE, _ROWS_PER_TILE)],
            vm_v, sem2)
        cp1.wait()
        cp2.wait()

        def per_row(r, accs):
            new = []
            for lvl in range(2):
                acc_s, acc_c = accs[2 * lvl], accs[2 * lvl + 1]
                base = lvl * _HALF
                margin = _MARGINS[lvl]

                c0 = sim_v[r, pl.ds(base, _LANES)]

                def max_step(k, mv):
                    return jnp.maximum(
                        mv, sim_v[r, pl.ds(base + k * _LANES, _LANES)])
                mv = lax.fori_loop(1, _NCHUNK, max_step, c0)
                m = jnp.max(mv)

                s0 = jnp.sum(jnp.where(iota == 0, c0, 0.0))
                s1 = jnp.sum(jnp.where(iota == 1, c0, 0.0))
                mm = m + margin

                fb0 = jnp.where(iota == 0, s1, s0) + margin
                negv = jnp.where(c0 < m, mm, fb0)
                vm0 = vm_v[r, pl.ds(base, _LANES)]
                acc_s = acc_s + jnp.maximum(negv - c0, 0.0) * vm0
                acc_c = acc_c + vm0

                fbs = s0 + margin

                def hinge_step(k, sc):
                    a_s, a_c = sc
                    off = base + k * _LANES
                    ck = sim_v[r, pl.ds(off, _LANES)]
                    negvk = jnp.where(ck < m, mm, fbs)
                    vmk = vm_v[r, pl.ds(off, _LANES)]
                    return (a_s + jnp.maximum(negvk - ck, 0.0) * vmk,
                            a_c + vmk)
                acc_s, acc_c = lax.fori_loop(
                    1, _NCHUNK, hinge_step, (acc_s, acc_c))
                new.extend([acc_s, acc_c])
            return tuple(new)

        accs = lax.fori_loop(0, _ROWS_PER_TILE, per_row,
                             (zeros, zeros, zeros, zeros))

        part = (jnp.where(iota == 0, jnp.sum(accs[0]), 0.0)
                + jnp.where(iota == 1, jnp.sum(accs[1]), 0.0)
                + jnp.where(iota == 2, jnp.sum(accs[2]), 0.0)
                + jnp.where(iota == 3, jnp.sum(accs[3]), 0.0))
        vec_v[...] = part
        pltpu.sync_copy(vec_v, shared.at[32 + sid])
        plsc.subcore_barrier()

        @pl.when(sid == 0)
        def _():
            pltpu.sync_copy(shared.at[pl.ds(32, _NSUB)], red_v)

            def red_step(i, acc):
                return acc + red_v[i, :]
            acc = lax.fori_loop(1, _NSUB, red_step, red_v[0, :])
            big_s1 = jnp.sum(jnp.where(iota == 0, acc, 0.0))
            big_c1 = jnp.sum(jnp.where(iota == 1, acc, 0.0))
            big_s2 = jnp.sum(jnp.where(iota == 2, acc, 0.0))
            big_c2 = jnp.sum(jnp.where(iota == 3, acc, 0.0))
            nv = (jnp.where(iota == 0, big_s1, 0.0)
                  + jnp.where(iota == 1, big_s2, 0.0))
            c2v = jnp.where(iota >= 0, big_c2, 0.0)
            dv = jnp.where(iota == 0, big_c1, jnp.maximum(c2v, 1.0))
            q = nv / dv
            gate = jnp.where(
                (iota == 0) | ((iota == 1) & (c2v >= 3.0)), 1.0, 0.0)
            loss = jnp.sum(q * gate)
            loss_v[0, :] = jnp.where(iota == 0, loss, 0.0)
            pltpu.sync_copy(loss_v, out_hbm)


def kernel(text_embed, label_embed, target):
    tgt = target.astype(jnp.int32)
    packed = pl.pallas_call(
        _sim_body,
        out_shape=jax.ShapeDtypeStruct((2 * _B, _L), jnp.float32),
    )(text_embed, label_embed, tgt)

    mine = pl.kernel(
        _sc_mine_body,
        out_type=jax.ShapeDtypeStruct((1, _LANES), jnp.float32),
        mesh=plsc.VectorSubcoreMesh(
            core_axis_name="c", subcore_axis_name="s",
            num_cores=1, num_subcores=_NSUB),
        scratch_types=[
            pltpu.VMEM((_ROWS_PER_TILE, _L), jnp.float32),
            pltpu.VMEM((_ROWS_PER_TILE, _L), jnp.float32),
            pltpu.VMEM((_LANES,), jnp.float32),
            pltpu.VMEM((1, _LANES), jnp.float32),
            pltpu.VMEM((_NSUB, _LANES), jnp.float32),
            pltpu.VMEM_SHARED((48, _LANES), jnp.float32),
            pltpu.SemaphoreType.DMA,
            pltpu.SemaphoreType.DMA,
        ],
        compiler_params=pltpu.CompilerParams(needs_layout_passes=False),
    )
    out = mine(packed)
    return out[0, 0]
